# Initial kernel scaffold; baseline (speedup 1.0000x reference)
import jax
import jax.numpy as jnp
from jax.experimental import pallas as pl


def kernel(x, table):
    raise NotImplementedError("write your pallas kernel here")


# speedup vs baseline: 1.6044x; 1.6044x over previous
import functools

import jax
import jax.numpy as jnp
from jax import lax
from jax.experimental import pallas as pl
from jax.experimental.pallas import tpu as pltpu
from jax.experimental.pallas import tpu_sc as plsc

B = 4096
L = 200
D = 32
DL = D * L
NC = 2
NS = 16
NW = NC * NS
SEQ_PER_W = B // NW
C0, C1 = 104, 96
UNROLL = 8


def _sc_body(x_hbm, table_hbm, emb_hbm, len_hbm,
             idx_v, rows0, rows1, outt0, outt1, len_v,
             gsem0, gsem1, osem0, osem1):
    wid = lax.axis_index("s") * NC + lax.axis_index("c")
    seq_base = wid * SEQ_PER_W

    pltpu.sync_copy(x_hbm.at[pl.ds(seq_base * L, SEQ_PER_W * L)], idx_v)

    lane = lax.iota(jnp.int32, 16)
    col_base = lane * L
    lane0 = lane == 0
    tail_mask = lane >= 8

    rows_bufs = (rows0, rows1)
    outt_bufs = (outt0, outt1)
    gsems = (gsem0, gsem1)
    osems = (osem0, osem1)

    def issue_gather(s_local, rows, gsem):
        base = s_local * L
        pltpu.async_copy(table_hbm.at[idx_v.at[pl.ds(base, C0)]],
                         rows.at[pl.ds(0, C0)], gsem)
        pltpu.async_copy(table_hbm.at[idx_v.at[pl.ds(base + C0, C1)]],
                         rows.at[pl.ds(C0, C1)], gsem)

    def transpose_seq(rows, outt):
        def tbody(t, carry):
            for j in range(UNROLL):
                l = t * UNROLL + j
                v0 = rows[l, pl.ds(0, 16)]
                v1 = rows[l, pl.ds(16, 16)]
                idx0 = col_base + l
                plsc.store_scatter(outt, [idx0], v0)
                plsc.store_scatter(outt, [idx0 + 16 * L], v1)
            return carry
        lax.fori_loop(0, L // UNROLL, tbody, 0)

    def count_seq(s_local):
        base = s_local * L
        cnt = jnp.zeros((16,), jnp.int32)
        for j in range(12):
            v = idx_v[pl.ds(base + j * 16, 16)]
            cnt = cnt + (v != 0).astype(jnp.int32)
        v = idx_v[pl.ds(base + 184, 16)]
        cnt = cnt + ((v != 0) & tail_mask).astype(jnp.int32)
        total = jnp.sum(cnt)
        plsc.store_scatter(len_v, [jnp.full((16,), s_local, jnp.int32)],
                           jnp.full((16,), total, jnp.int32), mask=lane0)

    issue_gather(0, rows0, gsem0)
    issue_gather(1, rows1, gsem1)

    def body(i, carry):
        for k in range(2):
            s = i * 2 + k
            rows, outt = rows_bufs[k], outt_bufs[k]
            gsem, osem = gsems[k], osems[k]

            pltpu.make_async_copy(table_hbm.at[pl.ds(0, L)], rows,
                                  gsem).wait()

            @pl.when(i > 0)
            def _():
                pltpu.make_async_copy(outt, emb_hbm.at[pl.ds(0, DL)],
                                      osem).wait()

            transpose_seq(rows, outt)
            count_seq(s)

            pltpu.async_copy(outt,
                             emb_hbm.at[pl.ds((seq_base + s) * DL, DL)],
                             osem)

            @pl.when(s + 2 < SEQ_PER_W)
            def _():
                issue_gather(s + 2, rows, gsem)
        return carry

    lax.fori_loop(0, SEQ_PER_W // 2, body, 0)

    for k in range(2):
        pltpu.make_async_copy(outt_bufs[k], emb_hbm.at[pl.ds(0, DL)],
                              osems[k]).wait()
    pltpu.sync_copy(len_v, len_hbm.at[pl.ds(seq_base, SEQ_PER_W)])


@functools.partial(
    pl.kernel,
    out_type=(jax.ShapeDtypeStruct((B * D * L,), jnp.float32),
              jax.ShapeDtypeStruct((B,), jnp.int32)),
    mesh=plsc.VectorSubcoreMesh(core_axis_name="c", subcore_axis_name="s",
                                num_cores=NC, num_subcores=NS),
    compiler_params=pltpu.CompilerParams(needs_layout_passes=False,
                                         use_tc_tiling_on_sc=False),
    scratch_types=[
        pltpu.VMEM((SEQ_PER_W * L,), jnp.int32),
        pltpu.VMEM((L, D), jnp.float32),
        pltpu.VMEM((L, D), jnp.float32),
        pltpu.VMEM((DL,), jnp.float32),
        pltpu.VMEM((DL,), jnp.float32),
        pltpu.VMEM((SEQ_PER_W,), jnp.int32),
        pltpu.SemaphoreType.DMA,
        pltpu.SemaphoreType.DMA,
        pltpu.SemaphoreType.DMA,
        pltpu.SemaphoreType.DMA,
    ],
)
def _embedding_sc(x_hbm, table_hbm, emb_hbm, len_hbm, *rest):
    _sc_body(x_hbm, table_hbm, emb_hbm, len_hbm, *rest)


def kernel(x, table):
    emb_flat, lengths = _embedding_sc(x.reshape(-1), table)
    return emb_flat.reshape(B, D, L), lengths
